# Initial kernel scaffold; baseline (speedup 1.0000x reference)
import jax
import jax.numpy as jnp
from jax.experimental import pallas as pl


def kernel(inputs, original_inputs, shared_wi0, shared_wi1, shared_wo, post_ff_scale_1, pre_ff_scale_2, post_ff_scale_2, router_scale, w_gate, expert_wi0, expert_wi1, expert_wo):
    raise NotImplementedError("write your pallas kernel here")


# speedup vs baseline: 1.0627x; 1.0627x over previous
import functools

import jax
import jax.numpy as jnp
from jax.experimental import pallas as pl
from jax.experimental.pallas import tpu as pltpu

B, S, D, F, E, K = 2, 2048, 2048, 2048, 8, 2
T = B * S
EPS = 1e-6
TM = 256
TMD = 512
NF = 4
FB = F // NF
NFB = 4
FBB = F // NFB
TB = 256
NT = T * K // TB + E
NSLOT = NT * TB


def _rms(x, scale, eps=EPS):
    var = jnp.mean(jnp.square(x), axis=-1, keepdims=True)
    y = x * jax.lax.rsqrt(var + eps)
    if scale is not None:
        y = y * scale
    return y


def _a_kernel(x_ref, orig_ref, wi0_ref, wi1_ref, wo_ref,
              ps1_ref, ps2_ref, rsc_ref, wg_ref,
              shared_ref, routed_in_ref, ti_ref, tw_ref, cnt_ref, prob_ref,
              acc_ref):
    i = pl.program_id(0)
    f = pl.program_id(1)
    nf = pl.num_programs(1)
    x = x_ref[...]

    @pl.when(f == 0)
    def _zero():
        acc_ref[...] = jnp.zeros_like(acc_ref)

    h0 = jnp.dot(x, wi0_ref[...], preferred_element_type=jnp.float32)
    h1 = jnp.dot(x, wi1_ref[...], preferred_element_type=jnp.float32)
    acc_ref[...] += jnp.dot(jax.nn.gelu(h0) * h1, wo_ref[...],
                            preferred_element_type=jnp.float32)

    @pl.when(f == nf - 1)
    def _fin():
        shared_ref[...] = _rms(acc_ref[...], ps1_ref[...])

    @pl.when(f == 0)
    def _router():
        og = orig_ref[...]
        var = jnp.mean(jnp.square(og), axis=-1, keepdims=True)
        inv = jax.lax.rsqrt(var + EPS)
        routed_in_ref[...] = og * inv * ps2_ref[...]
        gate_in = og * inv * (D ** -0.5) * rsc_ref[...]
        logits = jnp.dot(gate_in, wg_ref[...],
                         preferred_element_type=jnp.float32)
        lane = jax.lax.broadcasted_iota(jnp.int32, logits.shape, 1)
        v1 = jnp.max(logits, axis=-1, keepdims=True)
        i1 = jnp.min(jnp.where(logits == v1, lane, E), axis=-1, keepdims=True)
        masked = jnp.where(lane == i1, -jnp.inf, logits)
        v2 = jnp.max(masked, axis=-1, keepdims=True)
        i2 = jnp.min(jnp.where(masked == v2, lane, E), axis=-1, keepdims=True)
        m = jnp.maximum(v1, v2)
        e1 = jnp.exp(v1 - m)
        e2 = jnp.exp(v2 - m)
        den = e1 + e2
        w1, w2 = e1 / den, e2 / den
        sel1 = lane == i1
        sel2 = lane == i2
        ti_ref[...] = jnp.where(sel1, i1, 0) + jnp.where(sel2, i2, 0)
        tw_ref[...] = jnp.where(sel1, w1, 0.0) + jnp.where(sel2, w2, 0.0)
        probs = jax.nn.softmax(logits, axis=-1)
        onehot = (sel1 | sel2).astype(jnp.float32)
        cnt_ref[pl.ds(i, 1), :] = jnp.sum(onehot, axis=0, keepdims=True)
        prob_ref[pl.ds(i, 1), :] = jnp.sum(probs, axis=0, keepdims=True)


def _run_a(x, orig, wi0, wi1, wo, ps1, ps2, rsc, wg):
    grid = (T // TM, NF)
    out = pl.pallas_call(
        _a_kernel,
        grid=grid,
        in_specs=[
            pl.BlockSpec((TM, D), lambda i, f: (i, 0)),
            pl.BlockSpec((TM, D), lambda i, f: (i, 0)),
            pl.BlockSpec((D, FB), lambda i, f: (0, f)),
            pl.BlockSpec((D, FB), lambda i, f: (0, f)),
            pl.BlockSpec((FB, D), lambda i, f: (f, 0)),
            pl.BlockSpec((1, D), lambda i, f: (0, 0)),
            pl.BlockSpec((1, D), lambda i, f: (0, 0)),
            pl.BlockSpec((1, D), lambda i, f: (0, 0)),
            pl.BlockSpec((D, E), lambda i, f: (0, 0)),
        ],
        out_specs=[
            pl.BlockSpec((TM, D), lambda i, f: (i, 0)),
            pl.BlockSpec((TM, D), lambda i, f: (i, 0)),
            pl.BlockSpec((TM, E), lambda i, f: (i, 0)),
            pl.BlockSpec((TM, E), lambda i, f: (i, 0)),
            pl.BlockSpec((T // TM, E), lambda i, f: (0, 0)),
            pl.BlockSpec((T // TM, E), lambda i, f: (0, 0)),
        ],
        out_shape=[
            jax.ShapeDtypeStruct((T, D), jnp.float32),
            jax.ShapeDtypeStruct((T, D), jnp.float32),
            jax.ShapeDtypeStruct((T, E), jnp.int32),
            jax.ShapeDtypeStruct((T, E), jnp.float32),
            jax.ShapeDtypeStruct((T // TM, E), jnp.float32),
            jax.ShapeDtypeStruct((T // TM, E), jnp.float32),
        ],
        scratch_shapes=[pltpu.VMEM((TM, D), jnp.float32)],
    )(x, orig, wi0, wi1, wo, ps1.reshape(1, D), ps2.reshape(1, D),
      rsc.reshape(1, D), wg)
    return out


def _b_kernel(eid_ref, xg_ref, wi0_ref, wi1_ref, wo_ref, wr_ref, y_ref,
              acc_ref):
    f = pl.program_id(1)
    nf = pl.num_programs(1)

    @pl.when(f == 0)
    def _zero():
        acc_ref[...] = jnp.zeros_like(acc_ref)

    x = xg_ref[...]
    h0 = jnp.dot(x, wi0_ref[0], preferred_element_type=jnp.float32)
    h1 = jnp.dot(x, wi1_ref[0], preferred_element_type=jnp.float32)
    acc_ref[...] += jnp.dot(jax.nn.gelu(h0) * h1, wo_ref[0],
                            preferred_element_type=jnp.float32)

    @pl.when(f == nf - 1)
    def _fin():
        y_ref[...] = acc_ref[...] * wr_ref[0].reshape(TB, 1)


def _run_b(eids, xg, ewi0, ewi1, ewo, w_rows):
    grid_spec = pltpu.PrefetchScalarGridSpec(
        num_scalar_prefetch=1,
        grid=(NT, NFB),
        in_specs=[
            pl.BlockSpec((TB, D), lambda i, f, e: (i, 0)),
            pl.BlockSpec((1, D, FBB), lambda i, f, e: (e[i], 0, f)),
            pl.BlockSpec((1, D, FBB), lambda i, f, e: (e[i], 0, f)),
            pl.BlockSpec((1, FBB, D), lambda i, f, e: (e[i], f, 0)),
            pl.BlockSpec((1, 1, TB), lambda i, f, e: (i, 0, 0)),
        ],
        out_specs=pl.BlockSpec((TB, D), lambda i, f, e: (i, 0)),
        scratch_shapes=[pltpu.VMEM((TB, D), jnp.float32)],
    )
    return pl.pallas_call(
        _b_kernel,
        grid_spec=grid_spec,
        out_shape=jax.ShapeDtypeStruct((NSLOT, D), jnp.float32),
    )(eids, xg, ewi0, ewi1, ewo, w_rows.reshape(NT, 1, TB))


def _d_kernel(raw_ref, shared_ref, ps2_ref, out_ref):
    out_ref[...] = _rms(raw_ref[...], ps2_ref[...]) + shared_ref[...]


def _run_d(raw, shared, post2):
    return pl.pallas_call(
        _d_kernel,
        grid=(T // TMD,),
        in_specs=[
            pl.BlockSpec((TMD, D), lambda i: (i, 0)),
            pl.BlockSpec((TMD, D), lambda i: (i, 0)),
            pl.BlockSpec((1, D), lambda i: (0, 0)),
        ],
        out_specs=pl.BlockSpec((TMD, D), lambda i: (i, 0)),
        out_shape=jax.ShapeDtypeStruct((T, D), jnp.float32),
    )(raw, shared, post2.reshape(1, D))


def kernel(inputs, original_inputs, shared_wi0, shared_wi1, shared_wo,
           post_ff_scale_1, pre_ff_scale_2, post_ff_scale_2, router_scale,
           w_gate, expert_wi0, expert_wi1, expert_wo):
    x = inputs.reshape(T, D)
    orig = original_inputs.reshape(T, D)

    shared, routed_in, ti8, tw8, cnt_p, prob_p = _run_a(
        x, orig, shared_wi0, shared_wi1, shared_wo,
        post_ff_scale_1, pre_ff_scale_2, router_scale, w_gate)

    cnt = jnp.sum(cnt_p, axis=0)
    prob = jnp.sum(prob_p, axis=0)
    density = cnt / (T * K)
    density_prob = prob / T
    loss = jnp.sum(density * density_prob) * E

    lane = jnp.arange(E)[None, :]
    sel = tw8 > 0.0
    w1 = jnp.max(tw8, axis=-1)
    l1 = jnp.argmax(tw8, axis=-1)
    tw8_m = jnp.where(lane == l1[:, None], -1.0, tw8)
    w2 = jnp.max(tw8_m, axis=-1)
    l2 = jnp.argmax(tw8_m, axis=-1)
    del sel
    ti = jnp.stack([l1.astype(jnp.int32), l2.astype(jnp.int32)], axis=1)
    tw = jnp.stack([w1, w2], axis=1)

    e_flat = ti.reshape(-1)
    t_flat = jnp.repeat(jnp.arange(T, dtype=jnp.int32), K)
    w_flat = tw.reshape(-1)
    cnt_i = jnp.sum(jax.nn.one_hot(e_flat, E, dtype=jnp.int32), axis=0)
    gsize = ((cnt_i + TB - 1) // TB) * TB
    gstart = jnp.concatenate([jnp.zeros(1, jnp.int32), jnp.cumsum(gsize)])
    cstart = jnp.concatenate([jnp.zeros(1, jnp.int32),
                              jnp.cumsum(cnt_i)])[:-1]
    order = jnp.argsort(e_flat, stable=True)
    e_sorted = e_flat[order]
    j = jnp.arange(T * K, dtype=jnp.int32)
    slot_sorted = gstart[e_sorted] + (j - cstart[e_sorted])
    pos = jnp.zeros(T * K, jnp.int32).at[order].set(slot_sorted)
    tok_slot = jnp.zeros(NSLOT, jnp.int32).at[slot_sorted].set(t_flat[order])
    w_slot = jnp.zeros(NSLOT, jnp.float32).at[slot_sorted].set(w_flat[order])
    tile_base = jnp.arange(NT, dtype=jnp.int32) * TB
    eids = jnp.sum((tile_base[:, None] >= gstart[None, 1:E + 1])
                   .astype(jnp.int32), axis=1)

    xg = routed_in[tok_slot]
    y = _run_b(eids, xg, expert_wi0, expert_wi1, expert_wo, w_slot)
    routed_raw = y[pos.reshape(T, K)].sum(axis=1)

    out = _run_d(routed_raw, shared, post_ff_scale_2)
    return (out.reshape(B, S, D), loss,
            jnp.zeros((E,), jnp.float32))


# speedup vs baseline: 1.3051x; 1.2281x over previous
import functools

import jax
import jax.numpy as jnp
from jax import lax
from jax.experimental import pallas as pl
from jax.experimental.pallas import tpu as pltpu
from jax.experimental.pallas import tpu_sc as plsc

B, S, D, F, E, K = 2, 2048, 2048, 2048, 8, 2
T = B * S
EPS = 1e-6
TM = 256
TMD = 512
TMA2 = 512
NF = 4
FB = F // NF
NFB = 4
FBB = F // NFB
TB = 256
NT = T * K // TB + E
NSLOT = NT * TB

NC, NS, L = 2, 16, 16
NW = NC * NS
TPW_R = T // NS
TPW = T // NW

@functools.lru_cache(maxsize=1)
def _sc_mesh():
    return plsc.VectorSubcoreMesh(core_axis_name="c", subcore_axis_name="s")


def _rms(x, scale, eps=EPS):
    var = jnp.mean(jnp.square(x), axis=-1, keepdims=True)
    y = x * lax.rsqrt(var + eps)
    if scale is not None:
        y = y * scale
    return y


def _a_kernel(x_ref, orig_ref, wi0_ref, wi1_ref, wo_ref,
              ps1_ref, ps2_ref, rsc_ref, wg_ref,
              shared_ref, routed_in_ref, tw_ref, acc_ref):
    f = pl.program_id(1)
    nf = pl.num_programs(1)
    x = x_ref[...]

    @pl.when(f == 0)
    def _zero():
        acc_ref[...] = jnp.zeros_like(acc_ref)

    h0 = jnp.dot(x, wi0_ref[...], preferred_element_type=jnp.float32)
    h1 = jnp.dot(x, wi1_ref[...], preferred_element_type=jnp.float32)
    acc_ref[...] += jnp.dot(jax.nn.gelu(h0) * h1, wo_ref[...],
                            preferred_element_type=jnp.float32)

    @pl.when(f == nf - 1)
    def _fin():
        shared_ref[...] = _rms(acc_ref[...], ps1_ref[...])

    @pl.when(f == 0)
    def _router():
        og = orig_ref[...]
        var = jnp.mean(jnp.square(og), axis=-1, keepdims=True)
        inv = lax.rsqrt(var + EPS)
        routed_in_ref[...] = og * inv * ps2_ref[...]
        gate_in = og * inv * (D ** -0.5) * rsc_ref[...]
        logits = jnp.dot(gate_in, wg_ref[...],
                         preferred_element_type=jnp.float32)
        lane = lax.broadcasted_iota(jnp.int32, logits.shape, 1)
        v1 = jnp.max(logits, axis=-1, keepdims=True)
        i1 = jnp.min(jnp.where(logits == v1, lane, E), axis=-1, keepdims=True)
        masked = jnp.where(lane == i1, -jnp.inf, logits)
        v2 = jnp.max(masked, axis=-1, keepdims=True)
        i2 = jnp.min(jnp.where(masked == v2, lane, E), axis=-1, keepdims=True)
        m = jnp.maximum(v1, v2)
        e1 = jnp.exp(v1 - m)
        e2 = jnp.exp(v2 - m)
        den = e1 + e2
        sel1 = lane == i1
        sel2 = lane == i2
        tw_ref[...] = (jnp.where(sel1, e1 / den, 0.0)
                       + jnp.where(sel2, e2 / den, 0.0))


def _run_a(x, orig, wi0, wi1, wo, ps1, ps2, rsc, wg):
    return pl.pallas_call(
        _a_kernel,
        grid=(T // TM, NF),
        in_specs=[
            pl.BlockSpec((TM, D), lambda i, f: (i, 0)),
            pl.BlockSpec((TM, D), lambda i, f: (i, 0)),
            pl.BlockSpec((D, FB), lambda i, f: (0, f)),
            pl.BlockSpec((D, FB), lambda i, f: (0, f)),
            pl.BlockSpec((FB, D), lambda i, f: (f, 0)),
            pl.BlockSpec((1, D), lambda i, f: (0, 0)),
            pl.BlockSpec((1, D), lambda i, f: (0, 0)),
            pl.BlockSpec((1, D), lambda i, f: (0, 0)),
            pl.BlockSpec((D, E), lambda i, f: (0, 0)),
        ],
        out_specs=[
            pl.BlockSpec((TM, D), lambda i, f: (i, 0)),
            pl.BlockSpec((TM, D), lambda i, f: (i, 0)),
            pl.BlockSpec((TM, E), lambda i, f: (i, 0)),
        ],
        out_shape=[
            jax.ShapeDtypeStruct((T, D), jnp.float32),
            jax.ShapeDtypeStruct((T, D), jnp.float32),
            jax.ShapeDtypeStruct((T, E), jnp.float32),
        ],
        scratch_shapes=[pltpu.VMEM((TM, D), jnp.float32)],
    )(x, orig, wi0, wi1, wo, ps1.reshape(1, D), ps2.reshape(1, D),
      rsc.reshape(1, D), wg)


def _a2_kernel(ogt_ref, wgrt_ref, oh1_ref, oh2_ref,
               pos0_ref, pos1_ref, eids_ref, loss_ref,
               cnt_ref, prob_ref, run_ref, gst_ref):
    p = pl.program_id(0)
    i = pl.program_id(1)
    n = pl.num_programs(1)
    ogt = ogt_ref[...]
    var = jnp.mean(ogt * ogt, axis=0, keepdims=True)
    inv = lax.rsqrt(var + EPS)
    lg = jnp.dot(wgrt_ref[...], ogt, preferred_element_type=jnp.float32,
                 precision=lax.Precision.HIGHEST) * inv
    v1 = jnp.max(lg, axis=0, keepdims=True)
    oh1 = oh1_ref[...]
    oh2 = oh2_ref[...]

    @pl.when((p == 0) & (i == 0))
    def _zero():
        cnt_ref[...] = jnp.zeros_like(cnt_ref)
        prob_ref[...] = jnp.zeros_like(prob_ref)

    @pl.when(p == 0)
    def _phase0():
        pe = jnp.exp(lg - v1)
        probs = pe / jnp.sum(pe, axis=0, keepdims=True)
        li = lax.broadcasted_iota(jnp.int32, (E, 128), 1)
        lane0 = (li == 0).astype(jnp.float32)
        cnt_ref[...] += jnp.sum(oh1 + oh2, axis=1, keepdims=True) * lane0
        prob_ref[...] += jnp.sum(probs, axis=1, keepdims=True) * lane0

    @pl.when((p == 1) & (i == 0))
    def _offsets():
        tot = cnt_ref[:, 0:1]
        gsize = jnp.ceil(tot / TB) * TB
        se = lax.broadcasted_iota(jnp.int32, (E, E), 0)
        le = lax.broadcasted_iota(jnp.int32, (E, E), 1)
        tri8 = (le < se).astype(jnp.float32)
        gstart = jnp.dot(tri8, gsize, preferred_element_type=jnp.float32,
                         precision=lax.Precision.HIGHEST)
        gstart = jnp.floor(gstart + 0.5)
        gst_ref[...] = gstart * jnp.ones((E, 128), jnp.float32)
        run_ref[...] = jnp.zeros_like(run_ref)
        ends = gstart + gsize
        tb = (lax.broadcasted_iota(jnp.int32, (1, 128), 1)
              * TB).astype(jnp.float32)
        eid = jnp.sum((tb >= ends - 0.5).astype(jnp.int32), axis=0,
                      keepdims=True)
        eids_ref[...] = jnp.minimum(eid, E - 1)
        density = cnt_ref[...] / (T * K)
        dprob = prob_ref[...] / T
        lval = jnp.sum(density * dprob) * E
        loss_ref[...] = jnp.full((1, 128), lval, jnp.float32)

    @pl.when(p == 1)
    def _phase1():
        tt = lax.broadcasted_iota(jnp.int32, (TMA2, TMA2), 0)
        uu = lax.broadcasted_iota(jnp.int32, (TMA2, TMA2), 1)
        tri = (tt < uu).astype(jnp.float32)
        ohb = oh1 + oh2
        rank = jnp.dot(ohb, tri, preferred_element_type=jnp.float32,
                       precision=lax.Precision.HIGHEST)
        base = gst_ref[:, 0:1] + run_ref[:, 0:1] + rank
        pos0_ref[...] = (jnp.sum(oh1 * base, axis=0, keepdims=True)
                         + 0.5).astype(jnp.int32)
        pos1_ref[...] = (jnp.sum(oh2 * base, axis=0, keepdims=True)
                         + 0.5).astype(jnp.int32)
        li = lax.broadcasted_iota(jnp.int32, (E, 128), 1)
        lane0 = (li == 0).astype(jnp.float32)
        run_ref[...] += jnp.sum(ohb, axis=1, keepdims=True) * lane0


def _run_a2(ogt, wgrt, oh1t, oh2t):
    return pl.pallas_call(
        _a2_kernel,
        grid=(2, T // TMA2),
        in_specs=[
            pl.BlockSpec((D, TMA2), lambda p, i: (0, i)),
            pl.BlockSpec((E, D), lambda p, i: (0, 0)),
            pl.BlockSpec((E, TMA2), lambda p, i: (0, i)),
            pl.BlockSpec((E, TMA2), lambda p, i: (0, i)),
        ],
        out_specs=[
            pl.BlockSpec((1, TMA2), lambda p, i: (0, i)),
            pl.BlockSpec((1, TMA2), lambda p, i: (0, i)),
            pl.BlockSpec((1, 128), lambda p, i: (0, 0)),
            pl.BlockSpec((1, 128), lambda p, i: (0, 0)),
        ],
        out_shape=[
            jax.ShapeDtypeStruct((1, T), jnp.int32),
            jax.ShapeDtypeStruct((1, T), jnp.int32),
            jax.ShapeDtypeStruct((1, 128), jnp.int32),
            jax.ShapeDtypeStruct((1, 128), jnp.float32),
        ],
        scratch_shapes=[
            pltpu.VMEM((E, 128), jnp.float32),
            pltpu.VMEM((E, 128), jnp.float32),
            pltpu.VMEM((E, 128), jnp.float32),
            pltpu.VMEM((E, 128), jnp.float32),
        ],
    )(ogt, wgrt, oh1t, oh2t)


def _g_body(x_hbm, pos0_hbm, pos1_hbm, xg_hbm, idx0_v, idx1_v,
            buf0, buf1, sem0, sem1):
    wid = lax.axis_index("s") * NC + lax.axis_index("c")
    nch = TPW // L
    pltpu.sync_copy(pos0_hbm.at[pl.ds(wid * nch, nch)], idx0_v)
    pltpu.sync_copy(pos1_hbm.at[pl.ds(wid * nch, nch)], idx1_v)
    bufs = (buf0, buf1)
    sems = (sem0, sem1)
    cps = []
    for c in range(nch):
        b = c % 2
        if c >= 2:
            cps[c - 2][0].wait()
            cps[c - 2][1].wait()
        pltpu.sync_copy(x_hbm.at[pl.ds(wid * TPW + c * L, L)], bufs[b])
        cp0 = pltpu.async_copy(bufs[b], xg_hbm.at[idx0_v.at[c]], sems[b])
        cp1 = pltpu.async_copy(bufs[b], xg_hbm.at[idx1_v.at[c]], sems[b])
        cps.append((cp0, cp1))
    for c in (nch - 2, nch - 1):
        cps[c][0].wait()
        cps[c][1].wait()


def _run_g(x, pos0, pos1):
    kfn = functools.partial(
        pl.kernel,
        out_type=[jax.ShapeDtypeStruct((NSLOT, D), jnp.float32)],
        mesh=_sc_mesh(),
        scratch_types=[
            pltpu.VMEM((TPW // L, L), jnp.int32),
            pltpu.VMEM((TPW // L, L), jnp.int32),
            pltpu.VMEM((L, D), jnp.float32),
            pltpu.VMEM((L, D), jnp.float32),
            pltpu.SemaphoreType.DMA,
            pltpu.SemaphoreType.DMA,
        ],
    )
    return kfn(_g_body)(x, pos0, pos1)[0]


def _c_body(y_hbm, pos0_hbm, pos1_hbm, zg0_hbm, zg1_hbm, idx0_v, idx1_v,
            buf0, buf1, sem0, sem1):
    wid = lax.axis_index("s") * NC + lax.axis_index("c")
    nch = TPW // L
    pltpu.sync_copy(pos0_hbm.at[pl.ds(wid * nch, nch)], idx0_v)
    pltpu.sync_copy(pos1_hbm.at[pl.ds(wid * nch, nch)], idx1_v)
    for c in range(nch):
        cp0 = pltpu.async_copy(y_hbm.at[idx0_v.at[c]], buf0, sem0)
        cp1 = pltpu.async_copy(y_hbm.at[idx1_v.at[c]], buf1, sem1)
        cp0.wait()
        pltpu.sync_copy(buf0, zg0_hbm.at[pl.ds(wid * TPW + c * L, L)])
        cp1.wait()
        pltpu.sync_copy(buf1, zg1_hbm.at[pl.ds(wid * TPW + c * L, L)])


def _run_c(y, pos0, pos1):
    kfn = functools.partial(
        pl.kernel,
        out_type=[
            jax.ShapeDtypeStruct((T, D), jnp.float32),
            jax.ShapeDtypeStruct((T, D), jnp.float32),
        ],
        mesh=_sc_mesh(),
        scratch_types=[
            pltpu.VMEM((TPW // L, L), jnp.int32),
            pltpu.VMEM((TPW // L, L), jnp.int32),
            pltpu.VMEM((L, D), jnp.float32),
            pltpu.VMEM((L, D), jnp.float32),
            pltpu.SemaphoreType.DMA,
            pltpu.SemaphoreType.DMA,
        ],
    )
    return kfn(_c_body)(y, pos0, pos1)


def _b_kernel(eid_ref, xg_ref, wi0_ref, wi1_ref, wo_ref, y_ref, acc_ref):
    f = pl.program_id(1)
    nf = pl.num_programs(1)

    @pl.when(f == 0)
    def _zero():
        acc_ref[...] = jnp.zeros_like(acc_ref)

    x = xg_ref[...]
    h0 = jnp.dot(x, wi0_ref[0], preferred_element_type=jnp.float32)
    h1 = jnp.dot(x, wi1_ref[0], preferred_element_type=jnp.float32)
    acc_ref[...] += jnp.dot(jax.nn.gelu(h0) * h1, wo_ref[0],
                            preferred_element_type=jnp.float32)

    @pl.when(f == nf - 1)
    def _fin():
        y_ref[...] = acc_ref[...]


def _run_b(eids, xg, ewi0, ewi1, ewo):
    grid_spec = pltpu.PrefetchScalarGridSpec(
        num_scalar_prefetch=1,
        grid=(NT, NFB),
        in_specs=[
            pl.BlockSpec((TB, D), lambda i, f, e: (i, 0)),
            pl.BlockSpec((1, D, FBB), lambda i, f, e: (e[i], 0, f)),
            pl.BlockSpec((1, D, FBB), lambda i, f, e: (e[i], 0, f)),
            pl.BlockSpec((1, FBB, D), lambda i, f, e: (e[i], f, 0)),
        ],
        out_specs=pl.BlockSpec((TB, D), lambda i, f, e: (i, 0)),
        scratch_shapes=[pltpu.VMEM((TB, D), jnp.float32)],
    )
    return pl.pallas_call(
        _b_kernel,
        grid_spec=grid_spec,
        out_shape=jax.ShapeDtypeStruct((NSLOT, D), jnp.float32),
    )(eids, xg, ewi0, ewi1, ewo)


def _d_kernel(zg0_ref, zg1_ref, tw_ref, shared_ref, ps2_ref, out_ref):
    tw = tw_ref[...]
    w1 = jnp.max(tw, axis=-1, keepdims=True)
    w2 = jnp.sum(tw, axis=-1, keepdims=True) - w1
    raw = w1 * zg0_ref[...] + w2 * zg1_ref[...]
    out_ref[...] = _rms(raw, ps2_ref[...]) + shared_ref[...]


def _run_d(zg0, zg1, tw, shared, post2):
    return pl.pallas_call(
        _d_kernel,
        grid=(T // TMD,),
        in_specs=[
            pl.BlockSpec((TMD, D), lambda i: (i, 0)),
            pl.BlockSpec((TMD, D), lambda i: (i, 0)),
            pl.BlockSpec((TMD, E), lambda i: (i, 0)),
            pl.BlockSpec((TMD, D), lambda i: (i, 0)),
            pl.BlockSpec((1, D), lambda i: (0, 0)),
        ],
        out_specs=pl.BlockSpec((TMD, D), lambda i: (i, 0)),
        out_shape=jax.ShapeDtypeStruct((T, D), jnp.float32),
    )(zg0, zg1, tw, shared, post2.reshape(1, D))


def kernel(inputs, original_inputs, shared_wi0, shared_wi1, shared_wo,
           post_ff_scale_1, pre_ff_scale_2, post_ff_scale_2, router_scale,
           w_gate, expert_wi0, expert_wi1, expert_wo):
    x = inputs.reshape(T, D)
    orig = original_inputs.reshape(T, D)

    shared, routed_in, tw8 = _run_a(
        x, orig, shared_wi0, shared_wi1, shared_wo,
        post_ff_scale_1, pre_ff_scale_2, router_scale, w_gate)

    ogt = orig.T
    wgrt = (w_gate * (router_scale * (D ** -0.5))[:, None]).T
    l1 = jnp.argmax(tw8, axis=1)
    oh1t = jax.nn.one_hot(l1, E, dtype=jnp.float32).T
    oh2t = (tw8 > 0.0).astype(jnp.float32).T - oh1t
    pos0f, pos1f, eids128, loss = _run_a2(ogt, wgrt, oh1t, oh2t)

    pos0 = pos0f.reshape(T // L, L)
    pos1 = pos1f.reshape(T // L, L)
    xg = _run_g(routed_in, pos0, pos1)
    y = _run_b(eids128.reshape(128)[:NT], xg, expert_wi0, expert_wi1,
               expert_wo)
    zg0, zg1 = _run_c(y, pos0, pos1)
    out = _run_d(zg0, zg1, tw8, shared, post_ff_scale_2)
    return (out.reshape(B, S, D), loss[0, 0],
            jnp.zeros((E,), jnp.float32))


# speedup vs baseline: 1.4024x; 1.0745x over previous
import functools

import jax
import jax.numpy as jnp
from jax import lax
from jax.experimental import pallas as pl
from jax.experimental.pallas import tpu as pltpu
from jax.experimental.pallas import tpu_sc as plsc

B, S, D, F, E, K = 2, 2048, 2048, 2048, 8, 2
T = B * S
EPS = 1e-6
TM = 256
TMD = 512
TMA2 = 512
NF = 4
FB = F // NF
NFB = 4
FBB = F // NFB
TB = 256
NT = T * K // TB + E
NSLOT = NT * TB

NC, NS, L = 2, 16, 16
NW = NC * NS
TPW_R = T // NS
TPW = T // NW

@functools.lru_cache(maxsize=1)
def _sc_mesh():
    return plsc.VectorSubcoreMesh(core_axis_name="c", subcore_axis_name="s")


def _rms(x, scale, eps=EPS):
    var = jnp.mean(jnp.square(x), axis=-1, keepdims=True)
    y = x * lax.rsqrt(var + eps)
    if scale is not None:
        y = y * scale
    return y


def _a_kernel(x_ref, orig_ref, wi0_ref, wi1_ref, wo_ref,
              ps1_ref, ps2_ref, rsc_ref, wg_ref,
              shared_ref, routed_in_ref, tw_ref, acc_ref):
    f = pl.program_id(1)
    nf = pl.num_programs(1)
    x = x_ref[...]

    @pl.when(f == 0)
    def _zero():
        acc_ref[...] = jnp.zeros_like(acc_ref)

    xb = x.astype(jnp.bfloat16)
    h0 = jnp.dot(xb, wi0_ref[...], preferred_element_type=jnp.float32)
    h1 = jnp.dot(xb, wi1_ref[...], preferred_element_type=jnp.float32)
    g = (jax.nn.gelu(h0) * h1).astype(jnp.bfloat16)
    acc_ref[...] += jnp.dot(g, wo_ref[...],
                            preferred_element_type=jnp.float32)

    @pl.when(f == nf - 1)
    def _fin():
        shared_ref[...] = _rms(acc_ref[...], ps1_ref[...])

    @pl.when(f == 0)
    def _router():
        og = orig_ref[...]
        var = jnp.mean(jnp.square(og), axis=-1, keepdims=True)
        inv = lax.rsqrt(var + EPS)
        routed_in_ref[...] = og * inv * ps2_ref[...]
        gate_in = og * inv * (D ** -0.5) * rsc_ref[...]
        logits = jnp.dot(gate_in, wg_ref[...],
                         preferred_element_type=jnp.float32)
        lane = lax.broadcasted_iota(jnp.int32, logits.shape, 1)
        v1 = jnp.max(logits, axis=-1, keepdims=True)
        i1 = jnp.min(jnp.where(logits == v1, lane, E), axis=-1, keepdims=True)
        masked = jnp.where(lane == i1, -jnp.inf, logits)
        v2 = jnp.max(masked, axis=-1, keepdims=True)
        i2 = jnp.min(jnp.where(masked == v2, lane, E), axis=-1, keepdims=True)
        m = jnp.maximum(v1, v2)
        e1 = jnp.exp(v1 - m)
        e2 = jnp.exp(v2 - m)
        den = e1 + e2
        sel1 = lane == i1
        sel2 = lane == i2
        tw_ref[...] = (jnp.where(sel1, e1 / den, 0.0)
                       + jnp.where(sel2, e2 / den, 0.0))


def _run_a(x, orig, wi0, wi1, wo, ps1, ps2, rsc, wg):
    return pl.pallas_call(
        _a_kernel,
        grid=(T // TM, NF),
        in_specs=[
            pl.BlockSpec((TM, D), lambda i, f: (i, 0)),
            pl.BlockSpec((TM, D), lambda i, f: (i, 0)),
            pl.BlockSpec((D, FB), lambda i, f: (0, f)),
            pl.BlockSpec((D, FB), lambda i, f: (0, f)),
            pl.BlockSpec((FB, D), lambda i, f: (f, 0)),
            pl.BlockSpec((1, D), lambda i, f: (0, 0)),
            pl.BlockSpec((1, D), lambda i, f: (0, 0)),
            pl.BlockSpec((1, D), lambda i, f: (0, 0)),
            pl.BlockSpec((D, E), lambda i, f: (0, 0)),
        ],
        out_specs=[
            pl.BlockSpec((TM, D), lambda i, f: (i, 0)),
            pl.BlockSpec((TM, D), lambda i, f: (i, 0)),
            pl.BlockSpec((TM, E), lambda i, f: (i, 0)),
        ],
        out_shape=[
            jax.ShapeDtypeStruct((T, D), jnp.float32),
            jax.ShapeDtypeStruct((T, D), jnp.float32),
            jax.ShapeDtypeStruct((T, E), jnp.float32),
        ],
        scratch_shapes=[pltpu.VMEM((TM, D), jnp.float32)],
    )(x, orig, wi0, wi1, wo, ps1.reshape(1, D), ps2.reshape(1, D),
      rsc.reshape(1, D), wg)


def _a2_kernel(ogt_ref, wgrt_ref, oh1_ref, oh2_ref,
               pos0_ref, pos1_ref, eids_ref, loss_ref,
               cnt_ref, prob_ref, run_ref, gst_ref):
    p = pl.program_id(0)
    i = pl.program_id(1)
    n = pl.num_programs(1)
    ogt = ogt_ref[...]
    var = jnp.mean(ogt * ogt, axis=0, keepdims=True)
    inv = lax.rsqrt(var + EPS)
    lg = jnp.dot(wgrt_ref[...], ogt, preferred_element_type=jnp.float32,
                 precision=lax.Precision.HIGHEST) * inv
    v1 = jnp.max(lg, axis=0, keepdims=True)
    oh1 = oh1_ref[...]
    oh2 = oh2_ref[...]

    @pl.when((p == 0) & (i == 0))
    def _zero():
        cnt_ref[...] = jnp.zeros_like(cnt_ref)
        prob_ref[...] = jnp.zeros_like(prob_ref)

    @pl.when(p == 0)
    def _phase0():
        pe = jnp.exp(lg - v1)
        probs = pe / jnp.sum(pe, axis=0, keepdims=True)
        li = lax.broadcasted_iota(jnp.int32, (E, 128), 1)
        lane0 = (li == 0).astype(jnp.float32)
        cnt_ref[...] += jnp.sum(oh1 + oh2, axis=1, keepdims=True) * lane0
        prob_ref[...] += jnp.sum(probs, axis=1, keepdims=True) * lane0

    @pl.when((p == 1) & (i == 0))
    def _offsets():
        tot = cnt_ref[:, 0:1]
        gsize = jnp.ceil(tot / TB) * TB
        se = lax.broadcasted_iota(jnp.int32, (E, E), 0)
        le = lax.broadcasted_iota(jnp.int32, (E, E), 1)
        tri8 = (le < se).astype(jnp.float32)
        gstart = jnp.dot(tri8, gsize, preferred_element_type=jnp.float32,
                         precision=lax.Precision.HIGHEST)
        gstart = jnp.floor(gstart + 0.5)
        gst_ref[...] = gstart * jnp.ones((E, 128), jnp.float32)
        run_ref[...] = jnp.zeros_like(run_ref)
        ends = gstart + gsize
        tb = (lax.broadcasted_iota(jnp.int32, (1, 128), 1)
              * TB).astype(jnp.float32)
        eid = jnp.sum((tb >= ends - 0.5).astype(jnp.int32), axis=0,
                      keepdims=True)
        eids_ref[...] = jnp.minimum(eid, E - 1)
        density = cnt_ref[...] / (T * K)
        dprob = prob_ref[...] / T
        lval = jnp.sum(density * dprob) * E
        loss_ref[...] = jnp.full((1, 128), lval, jnp.float32)

    @pl.when(p == 1)
    def _phase1():
        tt = lax.broadcasted_iota(jnp.int32, (TMA2, TMA2), 0)
        uu = lax.broadcasted_iota(jnp.int32, (TMA2, TMA2), 1)
        tri = (tt < uu).astype(jnp.float32)
        ohb = oh1 + oh2
        rank = jnp.dot(ohb, tri, preferred_element_type=jnp.float32,
                       precision=lax.Precision.HIGHEST)
        base = gst_ref[:, 0:1] + run_ref[:, 0:1] + rank
        pos0_ref[...] = (jnp.sum(oh1 * base, axis=0, keepdims=True)
                         + 0.5).astype(jnp.int32)
        pos1_ref[...] = (jnp.sum(oh2 * base, axis=0, keepdims=True)
                         + 0.5).astype(jnp.int32)
        li = lax.broadcasted_iota(jnp.int32, (E, 128), 1)
        lane0 = (li == 0).astype(jnp.float32)
        run_ref[...] += jnp.sum(ohb, axis=1, keepdims=True) * lane0


def _run_a2(ogt, wgrt, oh1t, oh2t):
    return pl.pallas_call(
        _a2_kernel,
        grid=(2, T // TMA2),
        in_specs=[
            pl.BlockSpec((D, TMA2), lambda p, i: (0, i)),
            pl.BlockSpec((E, D), lambda p, i: (0, 0)),
            pl.BlockSpec((E, TMA2), lambda p, i: (0, i)),
            pl.BlockSpec((E, TMA2), lambda p, i: (0, i)),
        ],
        out_specs=[
            pl.BlockSpec((1, TMA2), lambda p, i: (0, i)),
            pl.BlockSpec((1, TMA2), lambda p, i: (0, i)),
            pl.BlockSpec((1, 128), lambda p, i: (0, 0)),
            pl.BlockSpec((1, 128), lambda p, i: (0, 0)),
        ],
        out_shape=[
            jax.ShapeDtypeStruct((1, T), jnp.int32),
            jax.ShapeDtypeStruct((1, T), jnp.int32),
            jax.ShapeDtypeStruct((1, 128), jnp.int32),
            jax.ShapeDtypeStruct((1, 128), jnp.float32),
        ],
        scratch_shapes=[
            pltpu.VMEM((E, 128), jnp.float32),
            pltpu.VMEM((E, 128), jnp.float32),
            pltpu.VMEM((E, 128), jnp.float32),
            pltpu.VMEM((E, 128), jnp.float32),
        ],
    )(ogt, wgrt, oh1t, oh2t)


def _g_body(x_hbm, pos0_hbm, pos1_hbm, xg_hbm, idx0_v, idx1_v,
            buf0, buf1, sem0, sem1):
    wid = lax.axis_index("s") * NC + lax.axis_index("c")
    nch = TPW // L
    pltpu.sync_copy(pos0_hbm.at[pl.ds(wid * nch, nch)], idx0_v)
    pltpu.sync_copy(pos1_hbm.at[pl.ds(wid * nch, nch)], idx1_v)
    bufs = (buf0, buf1)
    sems = (sem0, sem1)
    cps = []
    for c in range(nch):
        b = c % 2
        if c >= 2:
            cps[c - 2][0].wait()
            cps[c - 2][1].wait()
        pltpu.sync_copy(x_hbm.at[pl.ds(wid * TPW + c * L, L)], bufs[b])
        cp0 = pltpu.async_copy(bufs[b], xg_hbm.at[idx0_v.at[c]], sems[b])
        cp1 = pltpu.async_copy(bufs[b], xg_hbm.at[idx1_v.at[c]], sems[b])
        cps.append((cp0, cp1))
    for c in (nch - 2, nch - 1):
        cps[c][0].wait()
        cps[c][1].wait()


def _run_g(x, pos0, pos1):
    kfn = functools.partial(
        pl.kernel,
        out_type=[jax.ShapeDtypeStruct((NSLOT, D), jnp.float32)],
        mesh=_sc_mesh(),
        scratch_types=[
            pltpu.VMEM((TPW // L, L), jnp.int32),
            pltpu.VMEM((TPW // L, L), jnp.int32),
            pltpu.VMEM((L, D), jnp.float32),
            pltpu.VMEM((L, D), jnp.float32),
            pltpu.SemaphoreType.DMA,
            pltpu.SemaphoreType.DMA,
        ],
    )
    return kfn(_g_body)(x, pos0, pos1)[0]


def _c_body(y_hbm, pos0_hbm, pos1_hbm, zg0_hbm, zg1_hbm, idx0_v, idx1_v,
            buf0, buf1, sem0, sem1):
    wid = lax.axis_index("s") * NC + lax.axis_index("c")
    nch = TPW // L
    pltpu.sync_copy(pos0_hbm.at[pl.ds(wid * nch, nch)], idx0_v)
    pltpu.sync_copy(pos1_hbm.at[pl.ds(wid * nch, nch)], idx1_v)
    for c in range(nch):
        cp0 = pltpu.async_copy(y_hbm.at[idx0_v.at[c]], buf0, sem0)
        cp1 = pltpu.async_copy(y_hbm.at[idx1_v.at[c]], buf1, sem1)
        cp0.wait()
        pltpu.sync_copy(buf0, zg0_hbm.at[pl.ds(wid * TPW + c * L, L)])
        cp1.wait()
        pltpu.sync_copy(buf1, zg1_hbm.at[pl.ds(wid * TPW + c * L, L)])


def _run_c(y, pos0, pos1):
    kfn = functools.partial(
        pl.kernel,
        out_type=[
            jax.ShapeDtypeStruct((T, D), jnp.float32),
            jax.ShapeDtypeStruct((T, D), jnp.float32),
        ],
        mesh=_sc_mesh(),
        scratch_types=[
            pltpu.VMEM((TPW // L, L), jnp.int32),
            pltpu.VMEM((TPW // L, L), jnp.int32),
            pltpu.VMEM((L, D), jnp.float32),
            pltpu.VMEM((L, D), jnp.float32),
            pltpu.SemaphoreType.DMA,
            pltpu.SemaphoreType.DMA,
        ],
    )
    return kfn(_c_body)(y, pos0, pos1)


def _b_kernel(eid_ref, xg_ref, wi0_ref, wi1_ref, wo_ref, y_ref, acc_ref):
    f = pl.program_id(1)
    nf = pl.num_programs(1)

    @pl.when(f == 0)
    def _zero():
        acc_ref[...] = jnp.zeros_like(acc_ref)

    x = xg_ref[...].astype(jnp.bfloat16)
    h0 = jnp.dot(x, wi0_ref[0], preferred_element_type=jnp.float32)
    h1 = jnp.dot(x, wi1_ref[0], preferred_element_type=jnp.float32)
    g = (jax.nn.gelu(h0) * h1).astype(jnp.bfloat16)
    acc_ref[...] += jnp.dot(g, wo_ref[0],
                            preferred_element_type=jnp.float32)

    @pl.when(f == nf - 1)
    def _fin():
        y_ref[...] = acc_ref[...]


def _run_b(eids, xg, ewi0, ewi1, ewo):
    grid_spec = pltpu.PrefetchScalarGridSpec(
        num_scalar_prefetch=1,
        grid=(NT, NFB),
        in_specs=[
            pl.BlockSpec((TB, D), lambda i, f, e: (i, 0)),
            pl.BlockSpec((1, D, FBB), lambda i, f, e: (e[i], 0, f)),
            pl.BlockSpec((1, D, FBB), lambda i, f, e: (e[i], 0, f)),
            pl.BlockSpec((1, FBB, D), lambda i, f, e: (e[i], f, 0)),
        ],
        out_specs=pl.BlockSpec((TB, D), lambda i, f, e: (i, 0)),
        scratch_shapes=[pltpu.VMEM((TB, D), jnp.float32)],
    )
    return pl.pallas_call(
        _b_kernel,
        grid_spec=grid_spec,
        out_shape=jax.ShapeDtypeStruct((NSLOT, D), jnp.float32),
    )(eids, xg, ewi0, ewi1, ewo)


def _d_kernel(zg0_ref, zg1_ref, tw_ref, shared_ref, ps2_ref, out_ref):
    tw = tw_ref[...]
    w1 = jnp.max(tw, axis=-1, keepdims=True)
    w2 = jnp.sum(tw, axis=-1, keepdims=True) - w1
    raw = w1 * zg0_ref[...] + w2 * zg1_ref[...]
    out_ref[...] = _rms(raw, ps2_ref[...]) + shared_ref[...]


def _run_d(zg0, zg1, tw, shared, post2):
    return pl.pallas_call(
        _d_kernel,
        grid=(T // TMD,),
        in_specs=[
            pl.BlockSpec((TMD, D), lambda i: (i, 0)),
            pl.BlockSpec((TMD, D), lambda i: (i, 0)),
            pl.BlockSpec((TMD, E), lambda i: (i, 0)),
            pl.BlockSpec((TMD, D), lambda i: (i, 0)),
            pl.BlockSpec((1, D), lambda i: (0, 0)),
        ],
        out_specs=pl.BlockSpec((TMD, D), lambda i: (i, 0)),
        out_shape=jax.ShapeDtypeStruct((T, D), jnp.float32),
    )(zg0, zg1, tw, shared, post2.reshape(1, D))


def kernel(inputs, original_inputs, shared_wi0, shared_wi1, shared_wo,
           post_ff_scale_1, pre_ff_scale_2, post_ff_scale_2, router_scale,
           w_gate, expert_wi0, expert_wi1, expert_wo):
    x = inputs.reshape(T, D)
    orig = original_inputs.reshape(T, D)

    bf = jnp.bfloat16
    shared, routed_in, tw8 = _run_a(
        x, orig, shared_wi0.astype(bf), shared_wi1.astype(bf),
        shared_wo.astype(bf),
        post_ff_scale_1, pre_ff_scale_2, router_scale, w_gate)

    ogt = orig.T
    wgrt = (w_gate * (router_scale * (D ** -0.5))[:, None]).T
    l1 = jnp.argmax(tw8, axis=1)
    oh1t = jax.nn.one_hot(l1, E, dtype=jnp.float32).T
    oh2t = (tw8 > 0.0).astype(jnp.float32).T - oh1t
    pos0f, pos1f, eids128, loss = _run_a2(ogt, wgrt, oh1t, oh2t)

    pos0 = pos0f.reshape(T // L, L)
    pos1 = pos1f.reshape(T // L, L)
    xg = _run_g(routed_in, pos0, pos1)
    y = _run_b(eids128.reshape(128)[:NT], xg, expert_wi0.astype(bf),
               expert_wi1.astype(bf), expert_wo.astype(bf))
    zg0, zg1 = _run_c(y, pos0, pos1)
    out = _run_d(zg0, zg1, tw8, shared, post_ff_scale_2)
    return (out.reshape(B, S, D), loss[0, 0],
            jnp.zeros((E,), jnp.float32))
